# baseline (speedup 1.0000x reference)
import math

import jax
import jax.numpy as jnp
import numpy as np
from jax.experimental import pallas as pl
from jax.experimental.pallas import tpu as pltpu

EPS = 1e-06
S = 0.025
ALPHA = 0.98
DELTA = 2.0

CHUNK = 256
LANES = 128


def _pcen(xb, m):
    w = jax.lax.exp2(jnp.log(m + EPS) * np.float32(-ALPHA / math.log(2.0)))
    y = xb * w + DELTA
    return y * jax.lax.rsqrt(y) - np.float32(math.sqrt(DELTA))


def _pcen_kernel(x_ref, o_ref, l_scr, p_scr, m_scr):
    C = CHUNK
    t0 = pl.program_id(1) == 0

    @pl.when((pl.program_id(0) == 0) & t0)
    def _init():
        ln_a = np.float32(math.log(1.0 - S))
        ii = jax.lax.broadcasted_iota(jnp.int32, (C, C), 0)
        jj = jax.lax.broadcasted_iota(jnp.int32, (C, C), 1)
        di = (ii - jj).astype(jnp.float32)
        l_scr[...] = jnp.where(di >= 0.0, S * jnp.exp(di * ln_a), 0.0)
        ir = jax.lax.broadcasted_iota(jnp.int32, (C, LANES), 0).astype(jnp.float32)
        p_scr[...] = jnp.exp((ir + 1.0) * ln_a)

    for b2 in range(x_ref.shape[0]):
        m_prev = jnp.where(t0, x_ref[b2, 0:1, :], m_scr[b2:b2 + 1, :])
        for c in range(x_ref.shape[1] // C):
            xb = x_ref[b2, c * C:(c + 1) * C, :]
            m = jax.lax.dot_general(
                l_scr[...], xb, (((1,), (0,)), ((), ())),
                preferred_element_type=jnp.float32,
            ) + p_scr[...] * m_prev
            o_ref[b2, c * C:(c + 1) * C, :] = _pcen(xb, m)
            m_prev = m[C - 1:C, :]
        m_scr[b2:b2 + 1, :] = m_prev


def kernel(x):
    B, T, F = x.shape
    return pl.pallas_call(
        _pcen_kernel,
        grid=(B // 8, 4),
        in_specs=[pl.BlockSpec((8, T // 4, F), lambda b, t: (b, t, 0))],
        out_specs=pl.BlockSpec((8, T // 4, F), lambda b, t: (b, t, 0)),
        out_shape=jax.ShapeDtypeStruct((B, T, F), jnp.float32),
        scratch_shapes=[
            pltpu.VMEM((CHUNK, CHUNK), jnp.float32),
            pltpu.VMEM((CHUNK, F), jnp.float32),
            pltpu.VMEM((8, F), jnp.float32),
        ],
        compiler_params=pltpu.CompilerParams(
            dimension_semantics=("arbitrary", "arbitrary"),
        ),
    )(x)


# speedup vs baseline: 1.0054x; 1.0054x over previous
import math

import jax
import jax.numpy as jnp
import numpy as np
from jax.experimental import pallas as pl
from jax.experimental.pallas import tpu as pltpu

EPS = 1e-06
S = 0.025
ALPHA = 0.98
DELTA = 2.0

CHUNK = 256
LANES = 128


def _pcen(xb, m):
    w = jax.lax.exp2(jnp.log(m + EPS) * np.float32(-ALPHA / math.log(2.0)))
    y = xb * w + DELTA
    return y * jax.lax.rsqrt(y) - np.float32(math.sqrt(DELTA))


def _pcen_kernel(x_ref, o_ref, l_scr, p_scr, m_scr):
    C = CHUNK
    t0 = pl.program_id(1) == 0

    @pl.when((pl.program_id(0) == 0) & t0)
    def _init():
        ln_a = np.float32(math.log(1.0 - S))
        ii = jax.lax.broadcasted_iota(jnp.int32, (C, C), 0)
        jj = jax.lax.broadcasted_iota(jnp.int32, (C, C), 1)
        di = (ii - jj).astype(jnp.float32)
        l_scr[...] = jnp.where(di >= 0.0, S * jnp.exp(di * ln_a), 0.0)
        ir = jax.lax.broadcasted_iota(jnp.int32, (C, LANES), 0).astype(jnp.float32)
        p_scr[...] = jnp.exp((ir + 1.0) * ln_a)

    for b2 in range(x_ref.shape[0]):
        m_prev = jnp.where(t0, x_ref[b2, 0:1, :], m_scr[b2:b2 + 1, :])
        for c in range(x_ref.shape[1] // C):
            xb = x_ref[b2, c * C:(c + 1) * C, :]
            m = jax.lax.dot_general(
                l_scr[...], xb, (((1,), (0,)), ((), ())),
                preferred_element_type=jnp.float32,
            ) + p_scr[...] * m_prev
            o_ref[b2, c * C:(c + 1) * C, :] = _pcen(xb, m)
            m_prev = m[C - 1:C, :]
        m_scr[b2:b2 + 1, :] = m_prev


def kernel(x):
    B, T, F = x.shape
    return pl.pallas_call(
        _pcen_kernel,
        grid=(B // 4, 2),
        in_specs=[pl.BlockSpec((4, T // 2, F), lambda b, t: (b, t, 0))],
        out_specs=pl.BlockSpec((4, T // 2, F), lambda b, t: (b, t, 0)),
        out_shape=jax.ShapeDtypeStruct((B, T, F), jnp.float32),
        scratch_shapes=[
            pltpu.VMEM((CHUNK, CHUNK), jnp.float32),
            pltpu.VMEM((CHUNK, F), jnp.float32),
            pltpu.VMEM((8, F), jnp.float32),
        ],
        compiler_params=pltpu.CompilerParams(
            dimension_semantics=("arbitrary", "arbitrary"),
        ),
    )(x)
